# baseline (speedup 1.0000x reference)
import functools

import jax
import jax.numpy as jnp
from jax import lax
from jax.experimental import pallas as pl
from jax.experimental.pallas import tpu as pltpu
from jax.experimental.pallas import tpu_sc as plsc

L1_REG = 0.001
L2_REG = 0.001

N_IDX = 5 * 4096 * 26
D = 32
NW = 32
PER_W = N_IDX // NW
CHUNK = 128
N_CHUNKS = PER_W // CHUNK
G = 13
N_GROUPS = N_CHUNKS // G
GROUP_ROWS = G * CHUNK


def _gather_sc(W, Xf, out_shape):
    mesh = plsc.VectorSubcoreMesh(core_axis_name="c", subcore_axis_name="s")

    @functools.partial(
        pl.kernel,
        mesh=mesh,
        out_type=jax.ShapeDtypeStruct(out_shape, jnp.float32),
        scratch_types=[
            pltpu.VMEM((N_CHUNKS, CHUNK), jnp.int32),
            pltpu.VMEM((GROUP_ROWS, D), jnp.float32),
            pltpu.SemaphoreType.DMA,
        ],
        compiler_params=pltpu.CompilerParams(use_tc_tiling_on_sc=False),
    )
    def k(w_hbm, x_hbm, out_hbm, idx_v, rows_v, sem):
        nc = 2
        wid = lax.axis_index("s") * nc + lax.axis_index("c")
        pltpu.sync_copy(x_hbm.at[pl.ds(wid * N_CHUNKS, N_CHUNKS)], idx_v)

        def body(g, carry):
            cps = []
            for j in range(G):
                cps.append(pltpu.async_copy(
                    w_hbm.at[idx_v.at[g * G + j]],
                    rows_v.at[pl.ds(j * CHUNK, CHUNK)],
                    sem,
                ))
            for cp in cps:
                cp.wait()
            pltpu.sync_copy(
                rows_v,
                out_hbm.at[pl.ds(wid * PER_W + g * GROUP_ROWS, GROUP_ROWS)])
            return carry

        lax.fori_loop(0, N_GROUPS, body, 0)

    return k(W, Xf)


def _penalty_block(w_ref, out_ref):
    i = pl.program_id(0)
    x = w_ref[...]
    part = (L2_REG * 0.5) * jnp.sum(x * x) + L1_REG * jnp.sum(jnp.abs(x))

    @pl.when(i == 0)
    def _():
        out_ref[0, 0] = 0.0

    out_ref[0, 0] += part


def _penalty_tc(W):
    Wr = W.reshape(125000, 256)
    out = pl.pallas_call(
        _penalty_block,
        grid=(125,),
        in_specs=[pl.BlockSpec((1000, 256), lambda i: (i, 0))],
        out_specs=pl.BlockSpec(memory_space=pltpu.SMEM),
        out_shape=jax.ShapeDtypeStruct((1, 1), jnp.float32),
    )(Wr)
    return out[0, 0]


def kernel(X, W):
    n_samples, n_batch, input_dim = X.shape
    Xf = X.reshape(N_IDX // CHUNK, CHUNK)
    f_dim = input_dim * D
    rows = _gather_sc(W, Xf, (N_IDX, D))
    Net = rows.reshape(n_samples, n_batch, f_dim)
    penalty = _penalty_tc(W)
    return Net, penalty


# speedup vs baseline: 1.0008x; 1.0008x over previous
import functools

import jax
import jax.numpy as jnp
from jax import lax
from jax.experimental import pallas as pl
from jax.experimental.pallas import tpu as pltpu
from jax.experimental.pallas import tpu_sc as plsc

L1_REG = 0.001
L2_REG = 0.001

N_IDX = 5 * 4096 * 26
D = 32
NW = 32
PER_W = N_IDX // NW
CHUNK = 128
N_CHUNKS = PER_W // CHUNK
G = 13
N_GROUPS = N_CHUNKS // G
GROUP_ROWS = G * CHUNK


def _gather_sc(W, x_flat):
    mesh = plsc.VectorSubcoreMesh(core_axis_name="c", subcore_axis_name="s")

    @functools.partial(
        pl.kernel,
        mesh=mesh,
        out_type=jax.ShapeDtypeStruct((N_IDX, D), jnp.float32),
        scratch_types=[
            pltpu.VMEM((PER_W,), jnp.int32),
            pltpu.VMEM((GROUP_ROWS, D), jnp.float32),
            pltpu.SemaphoreType.DMA,
        ],
        compiler_params=pltpu.CompilerParams(use_tc_tiling_on_sc=False),
    )
    def k(w_hbm, x_hbm, out_hbm, idx_v, rows_v, sem):
        nc = 2
        wid = lax.axis_index("s") * nc + lax.axis_index("c")
        pltpu.sync_copy(x_hbm.at[pl.ds(wid * PER_W, PER_W)], idx_v)

        def body(g, carry):
            cps = []
            for j in range(G):
                cps.append(pltpu.async_copy(
                    w_hbm.at[idx_v.at[pl.ds((g * G + j) * CHUNK, CHUNK)]],
                    rows_v.at[pl.ds(j * CHUNK, CHUNK)],
                    sem,
                ))
            for cp in cps:
                cp.wait()
            pltpu.sync_copy(
                rows_v,
                out_hbm.at[pl.ds(wid * PER_W + g * GROUP_ROWS, GROUP_ROWS)])
            return carry

        lax.fori_loop(0, N_GROUPS, body, 0)

    return k(W, x_flat)


def _penalty_block(w_ref, out_ref):
    i = pl.program_id(0)
    x = w_ref[...]
    part = (L2_REG * 0.5) * jnp.sum(x * x) + L1_REG * jnp.sum(jnp.abs(x))

    @pl.when(i == 0)
    def _():
        out_ref[0, 0] = 0.0

    out_ref[0, 0] += part


def _penalty_tc(W):
    Wr = W.reshape(125000, 256)
    out = pl.pallas_call(
        _penalty_block,
        grid=(125,),
        in_specs=[pl.BlockSpec((1000, 256), lambda i: (i, 0))],
        out_specs=pl.BlockSpec(memory_space=pltpu.SMEM),
        out_shape=jax.ShapeDtypeStruct((1, 1), jnp.float32),
    )(Wr)
    return out[0, 0]


def kernel(X, W):
    n_samples, n_batch, input_dim = X.shape
    f_dim = input_dim * D
    rows = _gather_sc(W, X.reshape(-1))
    Net = rows.reshape(n_samples, n_batch, f_dim)
    penalty = _penalty_tc(W)
    return Net, penalty


# speedup vs baseline: 1.4645x; 1.4633x over previous
import functools

import jax
import jax.numpy as jnp
from jax import lax
from jax.experimental import pallas as pl
from jax.experimental.pallas import tpu as pltpu
from jax.experimental.pallas import tpu_sc as plsc

L1_REG = 0.001
L2_REG = 0.001

N_ROWS = 1000000
N_IDX = 5 * 4096 * 26
D = 32
NW = 32
PER_W = N_IDX // NW
CHUNK = 128
N_CHUNKS = PER_W // CHUNK
G = 13
N_GROUPS = N_CHUNKS // G
GROUP_ROWS = G * CHUNK

PEN_ROWS = N_ROWS // NW
PCH = 625
N_PCH = PEN_ROWS // PCH
PUNROLL = 5


def _reduce_chunk(buf, acc1, acc2):
    def rbody(t, carry):
        a1, a2 = carry
        for u in range(PUNROLL):
            r = t * PUNROLL + u
            for c in (0, 16):
                x = buf[r, pl.ds(c, 16)]
                a1 = a1 + x * x
                a2 = a2 + jnp.abs(x)
        return a1, a2

    return lax.fori_loop(0, PCH // PUNROLL, rbody, (acc1, acc2))


def _embed_sc(W, x_flat):
    mesh = plsc.VectorSubcoreMesh(core_axis_name="c", subcore_axis_name="s")

    @functools.partial(
        pl.kernel,
        mesh=mesh,
        out_type=(
            jax.ShapeDtypeStruct((N_IDX, D), jnp.float32),
            jax.ShapeDtypeStruct((NW, 2, 16), jnp.float32),
        ),
        scratch_types=[
            pltpu.VMEM((PER_W,), jnp.int32),
            pltpu.VMEM((GROUP_ROWS, D), jnp.float32),
            pltpu.VMEM((PCH, D), jnp.float32),
            pltpu.VMEM((PCH, D), jnp.float32),
            pltpu.VMEM((2, 16), jnp.float32),
            pltpu.SemaphoreType.DMA,
            pltpu.SemaphoreType.DMA,
            pltpu.SemaphoreType.DMA,
        ],
        compiler_params=pltpu.CompilerParams(use_tc_tiling_on_sc=False),
    )
    def k(w_hbm, x_hbm, out_hbm, pen_hbm,
          idx_v, rows_v, pen_a, pen_b, pacc_v, sem, psem_a, psem_b):
        nc = 2
        wid = lax.axis_index("s") * nc + lax.axis_index("c")

        pltpu.sync_copy(x_hbm.at[pl.ds(wid * PER_W, PER_W)], idx_v)

        def gbody(g, carry):
            cps = []
            for j in range(G):
                cps.append(pltpu.async_copy(
                    w_hbm.at[idx_v.at[pl.ds((g * G + j) * CHUNK, CHUNK)]],
                    rows_v.at[pl.ds(j * CHUNK, CHUNK)],
                    sem,
                ))
            for cp in cps:
                cp.wait()
            pltpu.sync_copy(
                rows_v,
                out_hbm.at[pl.ds(wid * PER_W + g * GROUP_ROWS, GROUP_ROWS)])
            return carry

        lax.fori_loop(0, N_GROUPS, gbody, 0)

        prow = wid * PEN_ROWS
        acc1 = jnp.zeros((16,), jnp.float32)
        acc2 = jnp.zeros((16,), jnp.float32)

        cp0 = pltpu.async_copy(w_hbm.at[pl.ds(prow, PCH)], pen_a, psem_a)
        cp0.wait()

        def pbody(t, carry):
            a1, a2 = carry
            c_even = 2 * t
            cpb = pltpu.async_copy(
                w_hbm.at[pl.ds(prow + (c_even + 1) * PCH, PCH)],
                pen_b, psem_b)
            a1, a2 = _reduce_chunk(pen_a, a1, a2)
            cpb.wait()

            @pl.when(c_even + 2 < N_PCH)
            def _():
                pltpu.make_async_copy(
                    w_hbm.at[pl.ds(prow + (c_even + 2) * PCH, PCH)],
                    pen_a, psem_a).start()

            a1, a2 = _reduce_chunk(pen_b, a1, a2)

            @pl.when(c_even + 2 < N_PCH)
            def _():
                pltpu.make_async_copy(
                    w_hbm.at[pl.ds(prow, PCH)], pen_a, psem_a).wait()

            return a1, a2

        acc1, acc2 = lax.fori_loop(0, N_PCH // 2, pbody, (acc1, acc2))

        pacc_v[0, pl.ds(0, 16)] = acc1
        pacc_v[1, pl.ds(0, 16)] = acc2
        pltpu.sync_copy(pacc_v, pen_hbm.at[wid])

    return k(W, x_flat)


def kernel(X, W):
    n_samples, n_batch, input_dim = X.shape
    f_dim = input_dim * D
    rows, pens = _embed_sc(W, X.reshape(-1))
    Net = rows.reshape(n_samples, n_batch, f_dim)
    penalty = (L2_REG * 0.5) * jnp.sum(pens[:, 0, :]) \
        + L1_REG * jnp.sum(pens[:, 1, :])
    return Net, penalty
